# Initial kernel scaffold; baseline (speedup 1.0000x reference)
import jax
import jax.numpy as jnp
from jax.experimental import pallas as pl


def kernel(species_id, ability_id, item_id, move_ids, type_ids, numerical_features, species_table, ability_table, item_table, move_table, type_table, Wa1, ba1, Wa2, ba2, Wt, bt, Wn, bn, g_n, be_n, Wo, bo, g_o, be_o):
    raise NotImplementedError("write your pallas kernel here")


# speedup vs baseline: 2.4524x; 2.4524x over previous
import functools

import jax
import jax.numpy as jnp
from jax import lax
from jax.experimental import pallas as pl
from jax.experimental.pallas import tpu as pltpu
from jax.experimental.pallas import tpu_sc as plsc

B = 16384
NC, NS = 2, 16
NW = NC * NS
ROWS_PER_W = B // NW
CHUNK = 128
NCHUNK = ROWS_PER_W // CHUNK


def _sc_gather(idx_all, sp_tab, ab_tab, it_tab, mv_tab, ty_tab):
    mesh = plsc.VectorSubcoreMesh(core_axis_name="c", subcore_axis_name="s")

    @functools.partial(
        pl.kernel,
        mesh=mesh,
        compiler_params=pltpu.CompilerParams(use_tc_tiling_on_sc=False),
        out_type=[
            jax.ShapeDtypeStruct((B, 64), jnp.float32),
            jax.ShapeDtypeStruct((B, 32), jnp.float32),
            jax.ShapeDtypeStruct((B, 32), jnp.float32),
            jax.ShapeDtypeStruct((4, B, 48), jnp.float32),
            jax.ShapeDtypeStruct((2, B, 16), jnp.float32),
        ],
        scratch_types=[
            pltpu.VMEM((9, CHUNK), jnp.int32),
            pltpu.VMEM((CHUNK, 64), jnp.float32),
            pltpu.VMEM((CHUNK, 32), jnp.float32),
            pltpu.VMEM((CHUNK, 32), jnp.float32),
            pltpu.VMEM((4, CHUNK, 48), jnp.float32),
            pltpu.VMEM((2, CHUNK, 16), jnp.float32),
            pltpu.SemaphoreType.DMA,
        ],
    )
    def k(idx_hbm, sp_hbm, ab_hbm, it_hbm, mv_hbm, ty_hbm,
          sp_out, ab_out, it_out, mv_out, ty_out,
          idx_v, sp_v, ab_v, it_v, mv_v, ty_v, sem):
        wid = lax.axis_index("s") * NC + lax.axis_index("c")

        def step(c, carry):
            base = wid * ROWS_PER_W + c * CHUNK
            pltpu.sync_copy(idx_hbm.at[:, pl.ds(base, CHUNK)], idx_v)
            cps = [
                pltpu.async_copy(sp_hbm.at[idx_v.at[0]], sp_v, sem),
                pltpu.async_copy(ab_hbm.at[idx_v.at[1]], ab_v, sem),
                pltpu.async_copy(it_hbm.at[idx_v.at[2]], it_v, sem),
            ]
            for j in range(4):
                cps.append(pltpu.async_copy(mv_hbm.at[idx_v.at[3 + j]],
                                            mv_v.at[j], sem))
            for j in range(2):
                cps.append(pltpu.async_copy(ty_hbm.at[idx_v.at[7 + j]],
                                            ty_v.at[j], sem))
            for cp in cps:
                cp.wait()
            pltpu.sync_copy(sp_v, sp_out.at[pl.ds(base, CHUNK)])
            pltpu.sync_copy(ab_v, ab_out.at[pl.ds(base, CHUNK)])
            pltpu.sync_copy(it_v, it_out.at[pl.ds(base, CHUNK)])
            pltpu.sync_copy(mv_v, mv_out.at[:, pl.ds(base, CHUNK)])
            pltpu.sync_copy(ty_v, ty_out.at[:, pl.ds(base, CHUNK)])
            return carry

        lax.fori_loop(0, NCHUNK, step, 0)

    return k(idx_all, sp_tab, ab_tab, it_tab, mv_tab, ty_tab)


def _dense_body(sp_ref, ab_ref, it_ref, mv_ref, ty_ref, num_ref,
                wa1_ref, ba1_ref, wa2_ref, wt_ref, bt_ref,
                wn_ref, bn_ref, gn_ref, ben_ref,
                wo_ref, bo_ref, go_ref, beo_ref, out_ref):
    f32 = jnp.float32
    hi = jax.lax.Precision.HIGHEST
    wa1 = wa1_ref[...]
    ba1 = ba1_ref[...]
    wa2 = wa2_ref[...]

    moves = [mv_ref[j] for j in range(4)]
    scores = []
    for m in moves:
        h = jnp.tanh(jnp.dot(m, wa1, precision=hi,
                             preferred_element_type=f32) + ba1)
        scores.append(jnp.sum(h * wa2, axis=-1, keepdims=True))
    mx = jnp.maximum(jnp.maximum(scores[0], scores[1]),
                     jnp.maximum(scores[2], scores[3]))
    es = [jnp.exp(s - mx) for s in scores]
    den = es[0] + es[1] + es[2] + es[3]
    move_emb = (es[0] * moves[0] + es[1] * moves[1]
                + es[2] * moves[2] + es[3] * moves[3]) / den

    ty = jnp.concatenate([ty_ref[0], ty_ref[1]], axis=-1)
    type_emb = jnp.dot(ty, wt_ref[...], precision=hi,
                       preferred_element_type=f32) + bt_ref[...]

    nh = jnp.dot(num_ref[...], wn_ref[...], precision=hi,
                 preferred_element_type=f32) + bn_ref[...]
    mu = jnp.mean(nh, axis=-1, keepdims=True)
    var = jnp.mean((nh - mu) ** 2, axis=-1, keepdims=True)
    nln = (nh - mu) * jax.lax.rsqrt(var + 1e-5) * gn_ref[...] + ben_ref[...]
    num_emb = jnp.maximum(nln, 0.0)

    comb = jnp.concatenate(
        [sp_ref[...], ab_ref[...], it_ref[...], move_emb, type_emb, num_emb],
        axis=-1)
    oh = jnp.dot(comb, wo_ref[...], precision=hi,
                 preferred_element_type=f32) + bo_ref[...]
    mu2 = jnp.mean(oh, axis=-1, keepdims=True)
    var2 = jnp.mean((oh - mu2) ** 2, axis=-1, keepdims=True)
    oln = (oh - mu2) * jax.lax.rsqrt(var2 + 1e-5) * go_ref[...] + beo_ref[...]
    out_ref[...] = jnp.maximum(oln, 0.0)


def _dense(sp_g, ab_g, it_g, mv_g, ty_g, numerical,
           Wa1, ba1, wa2, Wt, bt, Wn, bn, g_n, be_n, Wo, bo, g_o, be_o,
           blk=2048):
    grid = (B // blk,)
    full = lambda shape: pl.BlockSpec(shape, lambda i: (0,) * len(shape))
    return pl.pallas_call(
        _dense_body,
        grid=grid,
        in_specs=[
            pl.BlockSpec((blk, 64), lambda i: (i, 0)),
            pl.BlockSpec((blk, 32), lambda i: (i, 0)),
            pl.BlockSpec((blk, 32), lambda i: (i, 0)),
            pl.BlockSpec((4, blk, 48), lambda i: (0, i, 0)),
            pl.BlockSpec((2, blk, 16), lambda i: (0, i, 0)),
            pl.BlockSpec((blk, 20), lambda i: (i, 0)),
            full((48, 48)), full((1, 48)), full((1, 48)),
            full((32, 16)), full((1, 16)),
            full((20, 32)), full((1, 32)), full((1, 32)), full((1, 32)),
            full((224, 128)), full((1, 128)), full((1, 128)), full((1, 128)),
        ],
        out_specs=pl.BlockSpec((blk, 128), lambda i: (i, 0)),
        out_shape=jax.ShapeDtypeStruct((B, 128), jnp.float32),
    )(sp_g, ab_g, it_g, mv_g, ty_g, numerical,
      Wa1, ba1, wa2, Wt, bt, Wn, bn, g_n, be_n, Wo, bo, g_o, be_o)


def kernel(species_id, ability_id, item_id, move_ids, type_ids,
           numerical_features, species_table, ability_table, item_table,
           move_table, type_table, Wa1, ba1, Wa2, ba2, Wt, bt, Wn, bn,
           g_n, be_n, Wo, bo, g_o, be_o):
    idx_all = jnp.concatenate(
        [species_id[None].astype(jnp.int32),
         ability_id[None].astype(jnp.int32),
         item_id[None].astype(jnp.int32),
         move_ids.T.astype(jnp.int32),
         type_ids.T.astype(jnp.int32)], axis=0)

    sp_g, ab_g, it_g, mv_g, ty_g = _sc_gather(
        idx_all, species_table, ability_table, item_table,
        move_table, type_table)

    return _dense(sp_g, ab_g, it_g, mv_g, ty_g, numerical_features,
                  Wa1, ba1.reshape(1, 48), Wa2.reshape(1, 48),
                  Wt, bt.reshape(1, 16), Wn, bn.reshape(1, 32),
                  g_n.reshape(1, 32), be_n.reshape(1, 32),
                  Wo, bo.reshape(1, 128), g_o.reshape(1, 128),
                  be_o.reshape(1, 128))


# speedup vs baseline: 4.8870x; 1.9927x over previous
import functools

import jax
import jax.numpy as jnp
from jax import lax
from jax.experimental import pallas as pl
from jax.experimental.pallas import tpu as pltpu
from jax.experimental.pallas import tpu_sc as plsc

B = 16384
NC, NS = 2, 16
NW = NC * NS
ROWS_PER_W = B // NW
CHUNK = 128
NCHUNK = ROWS_PER_W // CHUNK


def _prepass_body(mt_ref, wa1_ref, ba1_ref, wa2_ref, tt_ref, wt_ref, bt_ref,
                  wo_ref, bo_ref,
                  me_ref, t0_ref, t1_ref, woa_ref, wob_ref, woc_ref, bo2_ref):
    f32 = jnp.float32
    mt = mt_ref[...]
    h = jnp.tanh(jnp.dot(mt, wa1_ref[...], preferred_element_type=f32)
                 + ba1_ref[...])
    s = jnp.sum(h * wa2_ref[...], axis=-1, keepdims=True)
    me_ref[...] = jnp.broadcast_to(s, (mt.shape[0], 16))

    tt = tt_ref[...]
    wt = wt_ref[...]
    t0_ref[...] = jnp.dot(tt, wt[0:16, :], preferred_element_type=f32)
    t1_ref[...] = jnp.dot(tt, wt[16:32, :], preferred_element_type=f32)

    wo = wo_ref[...]
    woa_ref[...] = wo[0:128, :]
    wob_ref[...] = jnp.concatenate(
        [wo[128:192, :], jnp.zeros((64, 128), f32)], axis=0)
    woc_ref[...] = wo[192:224, :]
    bo2_ref[...] = (bo_ref[...]
                    + jnp.dot(bt_ref[...], wo[176:192, :],
                              preferred_element_type=f32))


def _prepass(move_table, Wa1, ba1, wa2, type_table, Wt, bt, Wo, bo):
    f32 = jnp.float32
    out_shapes = [
        jax.ShapeDtypeStruct((920, 16), f32),
        jax.ShapeDtypeStruct((19, 16), f32),
        jax.ShapeDtypeStruct((19, 16), f32),
        jax.ShapeDtypeStruct((128, 128), f32),
        jax.ShapeDtypeStruct((128, 128), f32),
        jax.ShapeDtypeStruct((32, 128), f32),
        jax.ShapeDtypeStruct((1, 128), f32),
    ]
    return pl.pallas_call(
        _prepass_body,
        out_shape=out_shapes,
    )(move_table, Wa1, ba1, wa2, type_table, Wt, bt, Wo, bo)


def _sc_gather(species_id, ability_id, item_id, move_t, type_t,
               sp_tab, ab_tab, it_tab, mv_tab, sc_tab, t0_tab, t1_tab):
    mesh = plsc.VectorSubcoreMesh(core_axis_name="c", subcore_axis_name="s")

    @functools.partial(
        pl.kernel,
        mesh=mesh,
        compiler_params=pltpu.CompilerParams(use_tc_tiling_on_sc=False),
        out_type=[
            jax.ShapeDtypeStruct((B, 128), jnp.float32),
            jax.ShapeDtypeStruct((B, 128), jnp.float32),
        ],
        scratch_types=[
            pltpu.VMEM((9, CHUNK), jnp.int32),
            pltpu.VMEM((CHUNK, 64), jnp.float32),
            pltpu.VMEM((CHUNK, 32), jnp.float32),
            pltpu.VMEM((CHUNK, 32), jnp.float32),
            pltpu.VMEM((4, CHUNK, 48), jnp.float32),
            pltpu.VMEM((4, CHUNK, 16), jnp.float32),
            pltpu.VMEM((CHUNK, 16), jnp.float32),
            pltpu.VMEM((CHUNK, 16), jnp.float32),
            pltpu.VMEM((CHUNK, 128), jnp.float32),
            pltpu.SemaphoreType.DMA,
            pltpu.SemaphoreType.DMA,
        ],
    )
    def k(sid_hbm, aid_hbm, iid_hbm, mvt_hbm, tyt_hbm,
          sp_hbm, ab_hbm, it_hbm, mv_hbm, sc_hbm, t0_hbm, t1_hbm,
          sai_out, tail_out,
          idx_v, sp_v, ab_v, it_v, mv_v, sc_v, t0_v, t1_v, tl_v,
          sem, semw):
        wid = lax.axis_index("s") * NC + lax.axis_index("c")

        def zero_pad(r, carry):
            z = jnp.zeros((16,), jnp.float32)
            for kk in range(4):
                tl_v[r, pl.ds(64 + 16 * kk, 16)] = z
            return carry

        lax.fori_loop(0, CHUNK, zero_pad, 0)

        def step(c, carry):
            base = wid * ROWS_PER_W + c * CHUNK
            pltpu.sync_copy(sid_hbm.at[pl.ds(base, CHUNK)], idx_v.at[0])
            pltpu.sync_copy(aid_hbm.at[pl.ds(base, CHUNK)], idx_v.at[1])
            pltpu.sync_copy(iid_hbm.at[pl.ds(base, CHUNK)], idx_v.at[2])
            for j in range(4):
                pltpu.sync_copy(mvt_hbm.at[j, pl.ds(base, CHUNK)],
                                idx_v.at[3 + j])
            for j in range(2):
                pltpu.sync_copy(tyt_hbm.at[j, pl.ds(base, CHUNK)],
                                idx_v.at[7 + j])
            cps = [
                pltpu.async_copy(sp_hbm.at[idx_v.at[0]], sp_v, sem),
                pltpu.async_copy(ab_hbm.at[idx_v.at[1]], ab_v, sem),
                pltpu.async_copy(it_hbm.at[idx_v.at[2]], it_v, sem),
            ]
            for j in range(4):
                cps.append(pltpu.async_copy(mv_hbm.at[idx_v.at[3 + j]],
                                            mv_v.at[j], sem))
                cps.append(pltpu.async_copy(sc_hbm.at[idx_v.at[3 + j]],
                                            sc_v.at[j], sem))
            cps.append(pltpu.async_copy(t0_hbm.at[idx_v.at[7]], t0_v, sem))
            cps.append(pltpu.async_copy(t1_hbm.at[idx_v.at[8]], t1_v, sem))
            for cp in cps:
                cp.wait()

            def pool(r, carry):
                s0 = sc_v[0, r, :]
                s1 = sc_v[1, r, :]
                s2 = sc_v[2, r, :]
                s3 = sc_v[3, r, :]
                m = jnp.maximum(jnp.maximum(s0, s1), jnp.maximum(s2, s3))
                e0 = jnp.exp(s0 - m)
                e1 = jnp.exp(s1 - m)
                e2 = jnp.exp(s2 - m)
                e3 = jnp.exp(s3 - m)
                inv = 1.0 / (e0 + e1 + e2 + e3)
                w0 = e0 * inv
                w1 = e1 * inv
                w2 = e2 * inv
                w3 = e3 * inv
                for kk in range(3):
                    sl = pl.ds(16 * kk, 16)
                    tl_v[r, sl] = (w0 * mv_v[0, r, sl] + w1 * mv_v[1, r, sl]
                                   + w2 * mv_v[2, r, sl] + w3 * mv_v[3, r, sl])
                tl_v[r, pl.ds(48, 16)] = t0_v[r, :] + t1_v[r, :]
                return carry

            lax.fori_loop(0, CHUNK, pool, 0)

            wbs = [
                pltpu.async_copy(
                    sp_v, sai_out.at[pl.ds(base, CHUNK), pl.ds(0, 64)], semw),
                pltpu.async_copy(
                    ab_v, sai_out.at[pl.ds(base, CHUNK), pl.ds(64, 32)], semw),
                pltpu.async_copy(
                    it_v, sai_out.at[pl.ds(base, CHUNK), pl.ds(96, 32)], semw),
                pltpu.async_copy(tl_v, tail_out.at[pl.ds(base, CHUNK)], semw),
            ]
            for wb in wbs:
                wb.wait()
            return carry

        lax.fori_loop(0, NCHUNK, step, 0)

    return k(species_id, ability_id, item_id, move_t, type_t,
             sp_tab, ab_tab, it_tab, mv_tab, sc_tab, t0_tab, t1_tab)


def _main_body(sai_ref, tail_ref, num_ref, wn_ref, bn_ref, gn_ref, ben_ref,
               woa_ref, wob_ref, woc_ref, bo2_ref, go_ref, beo_ref, out_ref):
    f32 = jnp.float32
    nh = jnp.dot(num_ref[...], wn_ref[...],
                 preferred_element_type=f32) + bn_ref[...]
    mu = jnp.mean(nh, axis=-1, keepdims=True)
    var = jnp.mean((nh - mu) ** 2, axis=-1, keepdims=True)
    nln = (nh - mu) * jax.lax.rsqrt(var + 1e-5) * gn_ref[...] + ben_ref[...]
    num_emb = jnp.maximum(nln, 0.0)

    oh = (jnp.dot(sai_ref[...], woa_ref[...], preferred_element_type=f32)
          + jnp.dot(tail_ref[...], wob_ref[...], preferred_element_type=f32)
          + jnp.dot(num_emb, woc_ref[...], preferred_element_type=f32)
          + bo2_ref[...])
    mu2 = jnp.mean(oh, axis=-1, keepdims=True)
    var2 = jnp.mean((oh - mu2) ** 2, axis=-1, keepdims=True)
    oln = (oh - mu2) * jax.lax.rsqrt(var2 + 1e-5) * go_ref[...] + beo_ref[...]
    out_ref[...] = jnp.maximum(oln, 0.0)


def _main(sai, tail, numerical, Wn, bn, g_n, be_n, WoA, WoB, WoC, bo2,
          g_o, be_o, blk=2048):
    grid = (B // blk,)
    full = lambda shape: pl.BlockSpec(shape, lambda i: (0,) * len(shape))
    return pl.pallas_call(
        _main_body,
        grid=grid,
        in_specs=[
            pl.BlockSpec((blk, 128), lambda i: (i, 0)),
            pl.BlockSpec((blk, 128), lambda i: (i, 0)),
            pl.BlockSpec((blk, 20), lambda i: (i, 0)),
            full((20, 32)), full((32,)), full((32,)), full((32,)),
            full((128, 128)), full((128, 128)), full((32, 128)),
            full((1, 128)), full((128,)), full((128,)),
        ],
        out_specs=pl.BlockSpec((blk, 128), lambda i: (i, 0)),
        out_shape=jax.ShapeDtypeStruct((B, 128), jnp.float32),
    )(sai, tail, numerical, Wn, bn, g_n, be_n, WoA, WoB, WoC, bo2, g_o, be_o)


def kernel(species_id, ability_id, item_id, move_ids, type_ids,
           numerical_features, species_table, ability_table, item_table,
           move_table, type_table, Wa1, ba1, Wa2, ba2, Wt, bt, Wn, bn,
           g_n, be_n, Wo, bo, g_o, be_o):
    me, t0, t1, WoA, WoB, WoC, bo2 = _prepass(
        move_table, Wa1, ba1.reshape(1, 48), Wa2.reshape(1, 48),
        type_table, Wt, bt.reshape(1, 16), Wo, bo.reshape(1, 128))

    move_t = move_ids.T.astype(jnp.int32)
    type_t = type_ids.T.astype(jnp.int32)

    sai, tail = _sc_gather(
        species_id.astype(jnp.int32), ability_id.astype(jnp.int32),
        item_id.astype(jnp.int32), move_t, type_t,
        species_table, ability_table, item_table, move_table, me, t0, t1)

    return _main(sai, tail, numerical_features, Wn, bn, g_n, be_n,
                 WoA, WoB, WoC, bo2, g_o, be_o)


# speedup vs baseline: 5.6448x; 1.1551x over previous
import functools

import jax
import jax.numpy as jnp
from jax import lax
from jax.experimental import pallas as pl
from jax.experimental.pallas import tpu as pltpu
from jax.experimental.pallas import tpu_sc as plsc

B = 16384
NC, NS = 2, 16
NW = NC * NS
ROWS_PER_W = B // NW
CHUNK = 64
NCHUNK = ROWS_PER_W // CHUNK


def _prepass_body(mt_ref, wa1_ref, ba1_ref, wa2_ref, tt_ref, wt_ref, bt_ref,
                  wo_ref, bo_ref,
                  me_ref, t0_ref, t1_ref, woa_ref, wob_ref, woc_ref, bo2_ref):
    f32 = jnp.float32
    mt = mt_ref[...]
    h = jnp.tanh(jnp.dot(mt, wa1_ref[...], preferred_element_type=f32)
                 + ba1_ref[...])
    s = jnp.sum(h * wa2_ref[...], axis=-1, keepdims=True)
    me_ref[...] = jnp.broadcast_to(s, (mt.shape[0], 16))

    tt = tt_ref[...]
    wt = wt_ref[...]
    t0_ref[...] = jnp.dot(tt, wt[0:16, :], preferred_element_type=f32)
    t1_ref[...] = jnp.dot(tt, wt[16:32, :], preferred_element_type=f32)

    wo = wo_ref[...]
    woa_ref[...] = wo[0:128, :]
    wob_ref[...] = jnp.concatenate(
        [wo[128:192, :], jnp.zeros((64, 128), f32)], axis=0)
    woc_ref[...] = wo[192:224, :]
    bo2_ref[...] = (bo_ref[...]
                    + jnp.dot(bt_ref[...], wo[176:192, :],
                              preferred_element_type=f32))


def _prepass(move_table, Wa1, ba1, wa2, type_table, Wt, bt, Wo, bo):
    f32 = jnp.float32
    out_shapes = [
        jax.ShapeDtypeStruct((920, 16), f32),
        jax.ShapeDtypeStruct((19, 16), f32),
        jax.ShapeDtypeStruct((19, 16), f32),
        jax.ShapeDtypeStruct((128, 128), f32),
        jax.ShapeDtypeStruct((128, 128), f32),
        jax.ShapeDtypeStruct((32, 128), f32),
        jax.ShapeDtypeStruct((1, 128), f32),
    ]
    return pl.pallas_call(
        _prepass_body,
        out_shape=out_shapes,
    )(move_table, Wa1, ba1, wa2, type_table, Wt, bt, Wo, bo)


def _sc_gather(species_id, ability_id, item_id, move_t, type_t,
               sp_tab, ab_tab, it_tab, mv_tab, sc_tab, t0_tab, t1_tab):
    mesh = plsc.VectorSubcoreMesh(core_axis_name="c", subcore_axis_name="s")

    @functools.partial(
        pl.kernel,
        mesh=mesh,
        compiler_params=pltpu.CompilerParams(use_tc_tiling_on_sc=False),
        out_type=[
            jax.ShapeDtypeStruct((B, 128), jnp.float32),
            jax.ShapeDtypeStruct((B, 128), jnp.float32),
        ],
        scratch_types=[
            pltpu.VMEM((9, ROWS_PER_W), jnp.int32),
            pltpu.VMEM((2, CHUNK, 64), jnp.float32),
            pltpu.VMEM((2, CHUNK, 32), jnp.float32),
            pltpu.VMEM((2, CHUNK, 32), jnp.float32),
            pltpu.VMEM((2, 4, CHUNK, 48), jnp.float32),
            pltpu.VMEM((2, 4, CHUNK, 16), jnp.float32),
            pltpu.VMEM((2, CHUNK, 16), jnp.float32),
            pltpu.VMEM((2, CHUNK, 16), jnp.float32),
            pltpu.VMEM((2, CHUNK, 128), jnp.float32),
            pltpu.SemaphoreType.DMA,
            pltpu.SemaphoreType.DMA,
            pltpu.SemaphoreType.DMA,
            pltpu.SemaphoreType.DMA,
        ],
    )
    def k(sid_hbm, aid_hbm, iid_hbm, mvt_hbm, tyt_hbm,
          sp_hbm, ab_hbm, it_hbm, mv_hbm, sc_hbm, t0_hbm, t1_hbm,
          sai_out, tail_out,
          idx_v, sp_v, ab_v, it_v, mv_v, sc_v, t0_v, t1_v, tl_v,
          semi, semg, semw0, semw1):
        wid = lax.axis_index("s") * NC + lax.axis_index("c")
        wbase = wid * ROWS_PER_W

        icps = [
            pltpu.async_copy(sid_hbm.at[pl.ds(wbase, ROWS_PER_W)],
                             idx_v.at[0], semi),
            pltpu.async_copy(aid_hbm.at[pl.ds(wbase, ROWS_PER_W)],
                             idx_v.at[1], semi),
            pltpu.async_copy(iid_hbm.at[pl.ds(wbase, ROWS_PER_W)],
                             idx_v.at[2], semi),
        ]
        for j in range(4):
            icps.append(pltpu.async_copy(
                mvt_hbm.at[j, pl.ds(wbase, ROWS_PER_W)], idx_v.at[3 + j],
                semi))
        for j in range(2):
            icps.append(pltpu.async_copy(
                tyt_hbm.at[j, pl.ds(wbase, ROWS_PER_W)], idx_v.at[7 + j],
                semi))

        def zero_pad(r, carry):
            z = jnp.zeros((16,), jnp.float32)
            for sl in range(2):
                for kk in range(4):
                    tl_v[sl, r, pl.ds(64 + 16 * kk, 16)] = z
            return carry

        lax.fori_loop(0, CHUNK, zero_pad, 0)
        for cp in icps:
            cp.wait()

        def fire_gathers(c, s):
            off = c * CHUNK
            pltpu.async_copy(sp_hbm.at[idx_v.at[0, pl.ds(off, CHUNK)]],
                             sp_v.at[s], semg)
            pltpu.async_copy(ab_hbm.at[idx_v.at[1, pl.ds(off, CHUNK)]],
                             ab_v.at[s], semg)
            pltpu.async_copy(it_hbm.at[idx_v.at[2, pl.ds(off, CHUNK)]],
                             it_v.at[s], semg)
            for j in range(4):
                pltpu.async_copy(mv_hbm.at[idx_v.at[3 + j, pl.ds(off, CHUNK)]],
                                 mv_v.at[s, j], semg)
                pltpu.async_copy(sc_hbm.at[idx_v.at[3 + j, pl.ds(off, CHUNK)]],
                                 sc_v.at[s, j], semg)
            pltpu.async_copy(t0_hbm.at[idx_v.at[7, pl.ds(off, CHUNK)]],
                             t0_v.at[s], semg)
            pltpu.async_copy(t1_hbm.at[idx_v.at[8, pl.ds(off, CHUNK)]],
                             t1_v.at[s], semg)

        def drain_gathers(s):
            pltpu.make_async_copy(sp_hbm.at[idx_v.at[0, pl.ds(0, CHUNK)]],
                                  sp_v.at[s], semg).wait()
            pltpu.make_async_copy(ab_hbm.at[idx_v.at[1, pl.ds(0, CHUNK)]],
                                  ab_v.at[s], semg).wait()
            pltpu.make_async_copy(it_hbm.at[idx_v.at[2, pl.ds(0, CHUNK)]],
                                  it_v.at[s], semg).wait()
            for j in range(4):
                pltpu.make_async_copy(
                    mv_hbm.at[idx_v.at[3 + j, pl.ds(0, CHUNK)]],
                    mv_v.at[s, j], semg).wait()
                pltpu.make_async_copy(
                    sc_hbm.at[idx_v.at[3 + j, pl.ds(0, CHUNK)]],
                    sc_v.at[s, j], semg).wait()
            pltpu.make_async_copy(t0_hbm.at[idx_v.at[7, pl.ds(0, CHUNK)]],
                                  t0_v.at[s], semg).wait()
            pltpu.make_async_copy(t1_hbm.at[idx_v.at[8, pl.ds(0, CHUNK)]],
                                  t1_v.at[s], semg).wait()

        def fire_wb(c, s, semw):
            base = wbase + c * CHUNK
            pltpu.async_copy(
                sp_v.at[s], sai_out.at[pl.ds(base, CHUNK), pl.ds(0, 64)],
                semw)
            pltpu.async_copy(
                ab_v.at[s], sai_out.at[pl.ds(base, CHUNK), pl.ds(64, 32)],
                semw)
            pltpu.async_copy(
                it_v.at[s], sai_out.at[pl.ds(base, CHUNK), pl.ds(96, 32)],
                semw)
            pltpu.async_copy(tl_v.at[s], tail_out.at[pl.ds(base, CHUNK)],
                             semw)

        def drain_wb(s, semw):
            pltpu.make_async_copy(
                sp_v.at[s], sai_out.at[pl.ds(wbase, CHUNK), pl.ds(0, 64)],
                semw).wait()
            pltpu.make_async_copy(
                ab_v.at[s], sai_out.at[pl.ds(wbase, CHUNK), pl.ds(64, 32)],
                semw).wait()
            pltpu.make_async_copy(
                it_v.at[s], sai_out.at[pl.ds(wbase, CHUNK), pl.ds(96, 32)],
                semw).wait()
            pltpu.make_async_copy(tl_v.at[s], tail_out.at[pl.ds(wbase, CHUNK)],
                                  semw).wait()

        fire_gathers(0, 0)

        def step(c, carry):
            s = lax.rem(c, 2)
            drain_gathers(s)

            def pool(r, carry2):
                s0 = sc_v[s, 0, r, :]
                s1 = sc_v[s, 1, r, :]
                s2 = sc_v[s, 2, r, :]
                s3 = sc_v[s, 3, r, :]
                m = jnp.maximum(jnp.maximum(s0, s1), jnp.maximum(s2, s3))
                e0 = jnp.exp(s0 - m)
                e1 = jnp.exp(s1 - m)
                e2 = jnp.exp(s2 - m)
                e3 = jnp.exp(s3 - m)
                inv = 1.0 / (e0 + e1 + e2 + e3)
                w0 = e0 * inv
                w1 = e1 * inv
                w2 = e2 * inv
                w3 = e3 * inv
                for kk in range(3):
                    sl = pl.ds(16 * kk, 16)
                    tl_v[s, r, sl] = (
                        w0 * mv_v[s, 0, r, sl] + w1 * mv_v[s, 1, r, sl]
                        + w2 * mv_v[s, 2, r, sl] + w3 * mv_v[s, 3, r, sl])
                tl_v[s, r, pl.ds(48, 16)] = t0_v[s, r, :] + t1_v[s, r, :]
                return carry2

            lax.fori_loop(0, CHUNK, pool, 0)

            lax.cond(s == 0, lambda: fire_wb(c, 0, semw0),
                     lambda: fire_wb(c, 1, semw1))

            @pl.when(c + 1 < NCHUNK)
            def _():
                @pl.when(c >= 1)
                def _():
                    lax.cond(s == 0, lambda: drain_wb(1, semw1),
                             lambda: drain_wb(0, semw0))

                fire_gathers(c + 1, 1 - s)
            return carry

        lax.fori_loop(0, NCHUNK, step, 0)
        drain_wb(0, semw0)
        drain_wb(1, semw1)

    return k(species_id, ability_id, item_id, move_t, type_t,
             sp_tab, ab_tab, it_tab, mv_tab, sc_tab, t0_tab, t1_tab)


def _main_body(sai_ref, tail_ref, num_ref, wn_ref, bn_ref, gn_ref, ben_ref,
               woa_ref, wob_ref, woc_ref, bo2_ref, go_ref, beo_ref, out_ref):
    f32 = jnp.float32
    nh = jnp.dot(num_ref[...], wn_ref[...],
                 preferred_element_type=f32) + bn_ref[...]
    mu = jnp.mean(nh, axis=-1, keepdims=True)
    var = jnp.mean((nh - mu) ** 2, axis=-1, keepdims=True)
    nln = (nh - mu) * jax.lax.rsqrt(var + 1e-5) * gn_ref[...] + ben_ref[...]
    num_emb = jnp.maximum(nln, 0.0)

    oh = (jnp.dot(sai_ref[...], woa_ref[...], preferred_element_type=f32)
          + jnp.dot(tail_ref[...], wob_ref[...], preferred_element_type=f32)
          + jnp.dot(num_emb, woc_ref[...], preferred_element_type=f32)
          + bo2_ref[...])
    mu2 = jnp.mean(oh, axis=-1, keepdims=True)
    var2 = jnp.mean((oh - mu2) ** 2, axis=-1, keepdims=True)
    oln = (oh - mu2) * jax.lax.rsqrt(var2 + 1e-5) * go_ref[...] + beo_ref[...]
    out_ref[...] = jnp.maximum(oln, 0.0)


def _main(sai, tail, numerical, Wn, bn, g_n, be_n, WoA, WoB, WoC, bo2,
          g_o, be_o, blk=2048):
    grid = (B // blk,)
    full = lambda shape: pl.BlockSpec(shape, lambda i: (0,) * len(shape))
    return pl.pallas_call(
        _main_body,
        grid=grid,
        in_specs=[
            pl.BlockSpec((blk, 128), lambda i: (i, 0)),
            pl.BlockSpec((blk, 128), lambda i: (i, 0)),
            pl.BlockSpec((blk, 20), lambda i: (i, 0)),
            full((20, 32)), full((32,)), full((32,)), full((32,)),
            full((128, 128)), full((128, 128)), full((32, 128)),
            full((1, 128)), full((128,)), full((128,)),
        ],
        out_specs=pl.BlockSpec((blk, 128), lambda i: (i, 0)),
        out_shape=jax.ShapeDtypeStruct((B, 128), jnp.float32),
    )(sai, tail, numerical, Wn, bn, g_n, be_n, WoA, WoB, WoC, bo2, g_o, be_o)


def kernel(species_id, ability_id, item_id, move_ids, type_ids,
           numerical_features, species_table, ability_table, item_table,
           move_table, type_table, Wa1, ba1, Wa2, ba2, Wt, bt, Wn, bn,
           g_n, be_n, Wo, bo, g_o, be_o):
    me, t0, t1, WoA, WoB, WoC, bo2 = _prepass(
        move_table, Wa1, ba1.reshape(1, 48), Wa2.reshape(1, 48),
        type_table, Wt, bt.reshape(1, 16), Wo, bo.reshape(1, 128))

    move_t = move_ids.T.astype(jnp.int32)
    type_t = type_ids.T.astype(jnp.int32)

    sai, tail = _sc_gather(
        species_id.astype(jnp.int32), ability_id.astype(jnp.int32),
        item_id.astype(jnp.int32), move_t, type_t,
        species_table, ability_table, item_table, move_table, me, t0, t1)

    return _main(sai, tail, numerical_features, Wn, bn, g_n, be_n,
                 WoA, WoB, WoC, bo2, g_o, be_o)


# speedup vs baseline: 5.7367x; 1.0163x over previous
import functools

import jax
import jax.numpy as jnp
from jax import lax
from jax.experimental import pallas as pl
from jax.experimental.pallas import tpu as pltpu
from jax.experimental.pallas import tpu_sc as plsc

B = 16384
NC, NS = 2, 16
NW = NC * NS
ROWS_PER_W = B // NW
CHUNK = 64
NCHUNK = ROWS_PER_W // CHUNK


def _prepass_body(mt_ref, wa1_ref, ba1_ref, wa2_ref, tt_ref, wt_ref, bt_ref,
                  wo_ref, bo_ref,
                  comb_ref, woa_ref, wob_ref, woc_ref, bo2_ref):
    f32 = jnp.float32
    mt = mt_ref[...]
    h = jnp.tanh(jnp.dot(mt, wa1_ref[...], preferred_element_type=f32)
                 + ba1_ref[...])
    s = jnp.dot(h, wa2_ref[...], preferred_element_type=f32)
    scores = jnp.broadcast_to(s, (920, 16))

    tt = tt_ref[...]
    wt = wt_ref[...]
    t0 = jnp.dot(tt, wt[0:16, :], preferred_element_type=f32)
    t1 = jnp.dot(tt, wt[16:32, :], preferred_element_type=f32)
    zpad = jnp.zeros((5, 16), f32)
    comb_ref[...] = jnp.concatenate([scores, t0, zpad, t1, zpad], axis=0)

    wo = wo_ref[...]
    woa_ref[...] = wo[0:128, :]
    wob_ref[...] = jnp.concatenate(
        [wo[128:192, :], jnp.zeros((64, 128), f32)], axis=0)
    woc_ref[...] = wo[192:224, :]
    bo2_ref[...] = (bo_ref[...]
                    + jnp.dot(bt_ref[...], wo[176:192, :],
                              preferred_element_type=f32))


def _prepass(move_table, Wa1, ba1, Wa2, type_table, Wt, bt, Wo, bo):
    f32 = jnp.float32
    out_shapes = [
        jax.ShapeDtypeStruct((968, 16), f32),
        jax.ShapeDtypeStruct((128, 128), f32),
        jax.ShapeDtypeStruct((128, 128), f32),
        jax.ShapeDtypeStruct((32, 128), f32),
        jax.ShapeDtypeStruct((128,), f32),
    ]
    return pl.pallas_call(
        _prepass_body,
        out_shape=out_shapes,
    )(move_table, Wa1, ba1, Wa2, type_table, Wt, bt, Wo, bo)


def _sc_gather(species_id, ability_id, item_id, move_t, type_t,
               sp_tab, ab_tab, it_tab, mv_tab, sc_tab):
    mesh = plsc.VectorSubcoreMesh(core_axis_name="c", subcore_axis_name="s")

    @functools.partial(
        pl.kernel,
        mesh=mesh,
        compiler_params=pltpu.CompilerParams(use_tc_tiling_on_sc=False),
        out_type=[
            jax.ShapeDtypeStruct((B, 128), jnp.float32),
            jax.ShapeDtypeStruct((B, 128), jnp.float32),
        ],
        scratch_types=[
            pltpu.VMEM((9, ROWS_PER_W), jnp.int32),
            pltpu.VMEM((2, CHUNK, 64), jnp.float32),
            pltpu.VMEM((2, CHUNK, 32), jnp.float32),
            pltpu.VMEM((2, CHUNK, 32), jnp.float32),
            pltpu.VMEM((2, 4, CHUNK, 48), jnp.float32),
            pltpu.VMEM((2, 4, CHUNK, 16), jnp.float32),
            pltpu.VMEM((2, CHUNK, 16), jnp.float32),
            pltpu.VMEM((2, CHUNK, 16), jnp.float32),
            pltpu.VMEM((2, CHUNK, 128), jnp.float32),
            pltpu.SemaphoreType.DMA,
            pltpu.SemaphoreType.DMA,
            pltpu.SemaphoreType.DMA,
            pltpu.SemaphoreType.DMA,
        ],
    )
    def k(sid_hbm, aid_hbm, iid_hbm, mvt_hbm, tyt_hbm,
          sp_hbm, ab_hbm, it_hbm, mv_hbm, sc_hbm,
          sai_out, tail_out,
          idx_v, sp_v, ab_v, it_v, mv_v, sc_v, t0_v, t1_v, tl_v,
          semi, semg, semw0, semw1):
        wid = lax.axis_index("s") * NC + lax.axis_index("c")
        wbase = wid * ROWS_PER_W

        icps = [
            pltpu.async_copy(sid_hbm.at[pl.ds(wbase, ROWS_PER_W)],
                             idx_v.at[0], semi),
            pltpu.async_copy(aid_hbm.at[pl.ds(wbase, ROWS_PER_W)],
                             idx_v.at[1], semi),
            pltpu.async_copy(iid_hbm.at[pl.ds(wbase, ROWS_PER_W)],
                             idx_v.at[2], semi),
        ]
        for j in range(4):
            icps.append(pltpu.async_copy(
                mvt_hbm.at[j, pl.ds(wbase, ROWS_PER_W)], idx_v.at[3 + j],
                semi))
        for j in range(2):
            icps.append(pltpu.async_copy(
                tyt_hbm.at[j, pl.ds(wbase, ROWS_PER_W)], idx_v.at[7 + j],
                semi))

        def zero_pad(r, carry):
            z = jnp.zeros((16,), jnp.float32)
            for sl in range(2):
                for kk in range(4):
                    tl_v[sl, r, pl.ds(64 + 16 * kk, 16)] = z
            return carry

        lax.fori_loop(0, CHUNK, zero_pad, 0)
        for cp in icps:
            cp.wait()

        def fire_gathers(c, s):
            off = c * CHUNK
            pltpu.async_copy(sp_hbm.at[idx_v.at[0, pl.ds(off, CHUNK)]],
                             sp_v.at[s], semg)
            pltpu.async_copy(ab_hbm.at[idx_v.at[1, pl.ds(off, CHUNK)]],
                             ab_v.at[s], semg)
            pltpu.async_copy(it_hbm.at[idx_v.at[2, pl.ds(off, CHUNK)]],
                             it_v.at[s], semg)
            for j in range(4):
                pltpu.async_copy(mv_hbm.at[idx_v.at[3 + j, pl.ds(off, CHUNK)]],
                                 mv_v.at[s, j], semg)
                pltpu.async_copy(sc_hbm.at[idx_v.at[3 + j, pl.ds(off, CHUNK)]],
                                 sc_v.at[s, j], semg)
            pltpu.async_copy(sc_hbm.at[idx_v.at[7, pl.ds(off, CHUNK)]],
                             t0_v.at[s], semg)
            pltpu.async_copy(sc_hbm.at[idx_v.at[8, pl.ds(off, CHUNK)]],
                             t1_v.at[s], semg)

        def drain_gathers(s):
            pltpu.make_async_copy(sp_hbm.at[idx_v.at[0, pl.ds(0, CHUNK)]],
                                  sp_v.at[s], semg).wait()
            pltpu.make_async_copy(ab_hbm.at[idx_v.at[1, pl.ds(0, CHUNK)]],
                                  ab_v.at[s], semg).wait()
            pltpu.make_async_copy(it_hbm.at[idx_v.at[2, pl.ds(0, CHUNK)]],
                                  it_v.at[s], semg).wait()
            for j in range(4):
                pltpu.make_async_copy(
                    mv_hbm.at[idx_v.at[3 + j, pl.ds(0, CHUNK)]],
                    mv_v.at[s, j], semg).wait()
                pltpu.make_async_copy(
                    sc_hbm.at[idx_v.at[3 + j, pl.ds(0, CHUNK)]],
                    sc_v.at[s, j], semg).wait()
            pltpu.make_async_copy(sc_hbm.at[idx_v.at[7, pl.ds(0, CHUNK)]],
                                  t0_v.at[s], semg).wait()
            pltpu.make_async_copy(sc_hbm.at[idx_v.at[8, pl.ds(0, CHUNK)]],
                                  t1_v.at[s], semg).wait()

        def fire_wb(c, s, semw):
            base = wbase + c * CHUNK
            pltpu.async_copy(
                sp_v.at[s], sai_out.at[pl.ds(base, CHUNK), pl.ds(0, 64)],
                semw)
            pltpu.async_copy(
                ab_v.at[s], sai_out.at[pl.ds(base, CHUNK), pl.ds(64, 32)],
                semw)
            pltpu.async_copy(
                it_v.at[s], sai_out.at[pl.ds(base, CHUNK), pl.ds(96, 32)],
                semw)
            pltpu.async_copy(tl_v.at[s], tail_out.at[pl.ds(base, CHUNK)],
                             semw)

        def drain_wb(s, semw):
            pltpu.make_async_copy(
                sp_v.at[s], sai_out.at[pl.ds(wbase, CHUNK), pl.ds(0, 64)],
                semw).wait()
            pltpu.make_async_copy(
                ab_v.at[s], sai_out.at[pl.ds(wbase, CHUNK), pl.ds(64, 32)],
                semw).wait()
            pltpu.make_async_copy(
                it_v.at[s], sai_out.at[pl.ds(wbase, CHUNK), pl.ds(96, 32)],
                semw).wait()
            pltpu.make_async_copy(tl_v.at[s], tail_out.at[pl.ds(wbase, CHUNK)],
                                  semw).wait()

        fire_gathers(0, 0)

        def step(c, carry):
            s = lax.rem(c, 2)
            drain_gathers(s)

            @pl.when(c + 1 < NCHUNK)
            def _():
                @pl.when(c >= 1)
                def _():
                    lax.cond(s == 0, lambda: drain_wb(1, semw1),
                             lambda: drain_wb(0, semw0))

                fire_gathers(c + 1, 1 - s)

            def pool1(r):
                s0 = sc_v[s, 0, r, :]
                s1 = sc_v[s, 1, r, :]
                s2 = sc_v[s, 2, r, :]
                s3 = sc_v[s, 3, r, :]
                m = jnp.maximum(jnp.maximum(s0, s1), jnp.maximum(s2, s3))
                e0 = jnp.exp(s0 - m)
                e1 = jnp.exp(s1 - m)
                e2 = jnp.exp(s2 - m)
                e3 = jnp.exp(s3 - m)
                inv = 1.0 / (e0 + e1 + e2 + e3)
                w0 = e0 * inv
                w1 = e1 * inv
                w2 = e2 * inv
                w3 = e3 * inv
                for kk in range(3):
                    sl = pl.ds(16 * kk, 16)
                    tl_v[s, r, sl] = (
                        w0 * mv_v[s, 0, r, sl] + w1 * mv_v[s, 1, r, sl]
                        + w2 * mv_v[s, 2, r, sl] + w3 * mv_v[s, 3, r, sl])
                tl_v[s, r, pl.ds(48, 16)] = t0_v[s, r, :] + t1_v[s, r, :]

            def pool(r, carry2):
                pool1(2 * r)
                pool1(2 * r + 1)
                return carry2

            lax.fori_loop(0, CHUNK // 2, pool, 0)

            lax.cond(s == 0, lambda: fire_wb(c, 0, semw0),
                     lambda: fire_wb(c, 1, semw1))
            return carry

        lax.fori_loop(0, NCHUNK, step, 0)
        drain_wb(0, semw0)
        drain_wb(1, semw1)

    return k(species_id, ability_id, item_id, move_t, type_t,
             sp_tab, ab_tab, it_tab, mv_tab, sc_tab)


def _main_body(sai_ref, tail_ref, num_ref, wn_ref, bn_ref, gn_ref, ben_ref,
               woa_ref, wob_ref, woc_ref, bo2_ref, go_ref, beo_ref, out_ref):
    f32 = jnp.float32
    nh = jnp.dot(num_ref[...], wn_ref[...],
                 preferred_element_type=f32) + bn_ref[...]
    mu = jnp.mean(nh, axis=-1, keepdims=True)
    var = jnp.mean((nh - mu) ** 2, axis=-1, keepdims=True)
    nln = (nh - mu) * jax.lax.rsqrt(var + 1e-5) * gn_ref[...] + ben_ref[...]
    num_emb = jnp.maximum(nln, 0.0)

    oh = (jnp.dot(sai_ref[...], woa_ref[...], preferred_element_type=f32)
          + jnp.dot(tail_ref[...], wob_ref[...], preferred_element_type=f32)
          + jnp.dot(num_emb, woc_ref[...], preferred_element_type=f32)
          + bo2_ref[...])
    mu2 = jnp.mean(oh, axis=-1, keepdims=True)
    var2 = jnp.mean((oh - mu2) ** 2, axis=-1, keepdims=True)
    oln = (oh - mu2) * jax.lax.rsqrt(var2 + 1e-5) * go_ref[...] + beo_ref[...]
    out_ref[...] = jnp.maximum(oln, 0.0)


def _main(sai, tail, numerical, Wn, bn, g_n, be_n, WoA, WoB, WoC, bo2,
          g_o, be_o, blk=2048):
    grid = (B // blk,)
    full = lambda shape: pl.BlockSpec(shape, lambda i: (0,) * len(shape))
    return pl.pallas_call(
        _main_body,
        grid=grid,
        in_specs=[
            pl.BlockSpec((blk, 128), lambda i: (i, 0)),
            pl.BlockSpec((blk, 128), lambda i: (i, 0)),
            pl.BlockSpec((blk, 20), lambda i: (i, 0)),
            full((20, 32)), full((32,)), full((32,)), full((32,)),
            full((128, 128)), full((128, 128)), full((32, 128)),
            full((128,)), full((128,)), full((128,)),
        ],
        out_specs=pl.BlockSpec((blk, 128), lambda i: (i, 0)),
        out_shape=jax.ShapeDtypeStruct((B, 128), jnp.float32),
    )(sai, tail, numerical, Wn, bn, g_n, be_n, WoA, WoB, WoC, bo2, g_o, be_o)


def kernel(species_id, ability_id, item_id, move_ids, type_ids,
           numerical_features, species_table, ability_table, item_table,
           move_table, type_table, Wa1, ba1, Wa2, ba2, Wt, bt, Wn, bn,
           g_n, be_n, Wo, bo, g_o, be_o):
    comb, WoA, WoB, WoC, bo2 = _prepass(
        move_table, Wa1, ba1, Wa2, type_table, Wt, bt, Wo, bo)

    move_t = move_ids.T.astype(jnp.int32)
    type_t = (type_ids.T + jnp.array([[920], [944]], jnp.int32)).astype(
        jnp.int32)

    sai, tail = _sc_gather(
        species_id.astype(jnp.int32), ability_id.astype(jnp.int32),
        item_id.astype(jnp.int32), move_t, type_t,
        species_table, ability_table, item_table, move_table, comb)

    return _main(sai, tail, numerical_features, Wn, bn, g_n, be_n,
                 WoA, WoB, WoC, bo2, g_o, be_o)
